# Initial kernel scaffold; baseline (speedup 1.0000x reference)
import functools

import jax
import jax.numpy as jnp
from jax import lax
from jax.experimental import pallas as pl
from jax.experimental.pallas import tpu as pltpu
from jax.experimental.pallas import tpu_sc as plsc

_NC = 2
_NS = 16


@functools.lru_cache(maxsize=None)
def _make_sc_agg(N, E, W):
  nwk = _NC * _NS
  ew = E // nwk
  ch = 80
  assert ew % ch == 0 and E % nwk == 0
  nch = ew // ch
  rt = N // _NS
  assert N % _NS == 0 and (rt * W * 4) % 64 == 0

  mesh = plsc.VectorSubcoreMesh(
      core_axis_name="c", subcore_axis_name="s",
      num_cores=_NC, num_subcores=_NS)

  @functools.partial(
      pl.kernel,
      mesh=mesh,
      out_type=jax.ShapeDtypeStruct((_NC * N, W), jnp.float32),
      scratch_types=[
          pltpu.VMEM((ch,), jnp.int32),
          pltpu.VMEM((ch,), jnp.int32),
          pltpu.VMEM((ch, W), jnp.float32),
          pltpu.VMEM_SHARED((N, W), jnp.float32),
          pltpu.SemaphoreType.DMA,
      ],
  )
  def sc_agg(feat_hbm, src_hbm, dst_hbm, zero_hbm, out_hbm,
             src_v, dst_v, rows_v, acc, sem):
    c = lax.axis_index("c")
    s = lax.axis_index("s")
    r0 = s * rt
    pltpu.sync_copy(zero_hbm.at[pl.ds(r0, rt)], acc.at[pl.ds(r0, rt)])
    plsc.subcore_barrier()

    ebase = (c * _NS + s) * ew

    def body(i, carry):
      b = ebase + i * ch
      pltpu.sync_copy(src_hbm.at[pl.ds(b, ch)], src_v)
      pltpu.async_copy(feat_hbm.at[src_v], rows_v, sem).wait()
      pltpu.sync_copy(dst_hbm.at[pl.ds(b, ch)], dst_v)
      pltpu.sync_copy(rows_v, acc.at[dst_v], add=True)
      return carry

    lax.fori_loop(0, nch, body, 0)
    plsc.subcore_barrier()
    pltpu.sync_copy(acc.at[pl.ds(r0, rt)],
                    out_hbm.at[pl.ds(c * N + r0, rt)])

  return sc_agg


def _tc_layer1(x, p0, p1, w_self, w_neigh, b):
  n, d = x.shape
  wt = p0.shape[1]
  bn = 1000
  assert n % bn == 0

  def body(x_ref, p0_ref, p1_ref, ws_ref, wn_ref, b_ref, h_ref, dinv_ref):
    s = p0_ref[...] + p1_ref[...]
    dinv = 1.0 / jnp.maximum(s[:, d:d + 1], 1.0)
    agg = s[:, :d] * dinv
    h = (jnp.dot(x_ref[...], ws_ref[...], preferred_element_type=jnp.float32)
         + jnp.dot(agg, wn_ref[...], preferred_element_type=jnp.float32)
         + b_ref[...])
    h_ref[...] = jnp.maximum(h, 0.0)
    dinv_ref[...] = jnp.broadcast_to(dinv, (bn, d))

  return pl.pallas_call(
      body,
      grid=(n // bn,),
      in_specs=[
          pl.BlockSpec((bn, d), lambda i: (i, 0)),
          pl.BlockSpec((bn, wt), lambda i: (i, 0)),
          pl.BlockSpec((bn, wt), lambda i: (i, 0)),
          pl.BlockSpec((d, d), lambda i: (0, 0)),
          pl.BlockSpec((d, d), lambda i: (0, 0)),
          pl.BlockSpec((1, d), lambda i: (0, 0)),
      ],
      out_specs=[pl.BlockSpec((bn, d), lambda i: (i, 0)),
                 pl.BlockSpec((bn, d), lambda i: (i, 0))],
      out_shape=[jax.ShapeDtypeStruct((n, d), jnp.float32),
                 jax.ShapeDtypeStruct((n, d), jnp.float32)],
  )(x, p0, p1, w_self, w_neigh, b.reshape(1, d))


def _tc_layer2(h, q0, q1, dinv, w_self, w_neigh, b, w_fc, b_fc):
  n, d = h.shape
  co = w_fc.shape[1]
  bn = 1000
  assert n % bn == 0

  def body(h_ref, q0_ref, q1_ref, dinv_ref, ws_ref, wn_ref, b_ref,
           wfc_ref, bfc_ref, logits_ref, h2_ref):
    agg = (q0_ref[...] + q1_ref[...]) * dinv_ref[...]
    h2 = (jnp.dot(h_ref[...], ws_ref[...], preferred_element_type=jnp.float32)
          + jnp.dot(agg, wn_ref[...], preferred_element_type=jnp.float32)
          + b_ref[...])
    h2_ref[...] = h2
    logits_ref[...] = (
        jnp.dot(h2, wfc_ref[...], preferred_element_type=jnp.float32)
        + bfc_ref[...])

  return pl.pallas_call(
      body,
      grid=(n // bn,),
      in_specs=[
          pl.BlockSpec((bn, d), lambda i: (i, 0)),
          pl.BlockSpec((bn, d), lambda i: (i, 0)),
          pl.BlockSpec((bn, d), lambda i: (i, 0)),
          pl.BlockSpec((bn, d), lambda i: (i, 0)),
          pl.BlockSpec((d, d), lambda i: (0, 0)),
          pl.BlockSpec((d, d), lambda i: (0, 0)),
          pl.BlockSpec((1, d), lambda i: (0, 0)),
          pl.BlockSpec((d, co), lambda i: (0, 0)),
          pl.BlockSpec((1, co), lambda i: (0, 0)),
      ],
      out_specs=[pl.BlockSpec((bn, co), lambda i: (i, 0)),
                 pl.BlockSpec((bn, d), lambda i: (i, 0))],
      out_shape=[jax.ShapeDtypeStruct((n, co), jnp.float32),
                 jax.ShapeDtypeStruct((n, d), jnp.float32)],
  )(h, q0, q1, dinv, w_self, w_neigh, b.reshape(1, d), w_fc,
    b_fc.reshape(1, co))


def kernel(x, edge_index, W_self1, W_neigh1, b1, W_self2, W_neigh2, b2,
           W_fc, b_fc):
  n, d = x.shape
  e = edge_index.shape[1]
  w1 = d + 16

  src = edge_index[0].astype(jnp.int32)
  dst = edge_index[1].astype(jnp.int32)
  xpad = jnp.concatenate(
      [x, jnp.ones((n, 1), jnp.float32), jnp.zeros((n, 15), jnp.float32)],
      axis=1)
  zeros1 = jnp.zeros((n, w1), jnp.float32)
  zeros2 = jnp.zeros((n, d), jnp.float32)

  p = _make_sc_agg(n, e, w1)(xpad, src, dst, zeros1)
  h, dinv = _tc_layer1(x, p[:n], p[n:], W_self1, W_neigh1, b1)
  q = _make_sc_agg(n, e, d)(h, src, dst, zeros2)
  logits, h2 = _tc_layer2(h, q[:n], q[n:], dinv, W_self2, W_neigh2, b2,
                          W_fc, b_fc)
  return (logits, h2)


# speedup vs baseline: 5.0116x; 5.0116x over previous
import functools

import jax
import jax.numpy as jnp
from jax import lax
from jax.experimental import pallas as pl
from jax.experimental.pallas import tpu as pltpu
from jax.experimental.pallas import tpu_sc as plsc

_NC = 2
_NS = 16


@functools.lru_cache(maxsize=None)
def _make_sc_agg(N, E, W):
  nwk = _NC * _NS
  ew = E // nwk
  ch = 80
  assert ew % ch == 0 and E % nwk == 0
  nch = ew // ch
  rt = N // _NS
  assert N % _NS == 0 and rt % 8 == 0 and (rt * W * 4) % 64 == 0

  mesh = plsc.VectorSubcoreMesh(
      core_axis_name="c", subcore_axis_name="s",
      num_cores=_NC, num_subcores=_NS)

  @functools.partial(
      pl.kernel,
      mesh=mesh,
      compiler_params=pltpu.CompilerParams(use_tc_tiling_on_sc=False),
      out_type=jax.ShapeDtypeStruct((_NC * N, W), jnp.float32),
      scratch_types=[
          pltpu.VMEM((ch,), jnp.int32),
          pltpu.VMEM((ch,), jnp.int32),
          pltpu.VMEM((ch, W), jnp.float32),
          pltpu.VMEM_SHARED((N, W), jnp.float32),
          pltpu.SemaphoreType.DMA,
      ],
  )
  def sc_agg(feat_hbm, src_hbm, dst_hbm, zero_hbm, out_hbm,
             src_v, dst_v, rows_v, acc, sem):
    c = lax.axis_index("c")
    s = lax.axis_index("s")
    r0 = s * rt
    pltpu.sync_copy(zero_hbm.at[pl.ds(r0, rt)], acc.at[pl.ds(r0, rt)])
    plsc.subcore_barrier()

    ebase = (c * _NS + s) * ew

    def body(i, carry):
      b = ebase + i * ch
      pltpu.sync_copy(src_hbm.at[pl.ds(b, ch)], src_v)
      pltpu.async_copy(feat_hbm.at[src_v], rows_v, sem).wait()
      pltpu.sync_copy(dst_hbm.at[pl.ds(b, ch)], dst_v)
      pltpu.sync_copy(rows_v, acc.at[dst_v], add=True)
      return carry

    lax.fori_loop(0, nch, body, 0)
    plsc.subcore_barrier()
    pltpu.sync_copy(acc.at[pl.ds(r0, rt)],
                    out_hbm.at[pl.ds(c * N + r0, rt)])

  return sc_agg


def _tc_layer1(x, p0, p1, w_self, w_neigh, b):
  n, d = x.shape
  wt = p0.shape[1]
  bn = 1000
  assert n % bn == 0

  def body(x_ref, p0_ref, p1_ref, ws_ref, wn_ref, b_ref, h_ref, dinv_ref):
    s = p0_ref[...] + p1_ref[...]
    dinv = 1.0 / jnp.maximum(s[:, d:d + 1], 1.0)
    agg = s[:, :d] * dinv
    h = (jnp.dot(x_ref[...], ws_ref[...], preferred_element_type=jnp.float32)
         + jnp.dot(agg, wn_ref[...], preferred_element_type=jnp.float32)
         + b_ref[...])
    h_ref[...] = jnp.maximum(h, 0.0)
    dinv_ref[...] = jnp.broadcast_to(dinv, (bn, d))

  return pl.pallas_call(
      body,
      grid=(n // bn,),
      in_specs=[
          pl.BlockSpec((bn, d), lambda i: (i, 0)),
          pl.BlockSpec((bn, wt), lambda i: (i, 0)),
          pl.BlockSpec((bn, wt), lambda i: (i, 0)),
          pl.BlockSpec((d, d), lambda i: (0, 0)),
          pl.BlockSpec((d, d), lambda i: (0, 0)),
          pl.BlockSpec((1, d), lambda i: (0, 0)),
      ],
      out_specs=[pl.BlockSpec((bn, d), lambda i: (i, 0)),
                 pl.BlockSpec((bn, d), lambda i: (i, 0))],
      out_shape=[jax.ShapeDtypeStruct((n, d), jnp.float32),
                 jax.ShapeDtypeStruct((n, d), jnp.float32)],
  )(x, p0, p1, w_self, w_neigh, b.reshape(1, d))


def _tc_layer2(h, q0, q1, dinv, w_self, w_neigh, b, w_fc, b_fc):
  n, d = h.shape
  co = w_fc.shape[1]
  bn = 1000
  assert n % bn == 0

  def body(h_ref, q0_ref, q1_ref, dinv_ref, ws_ref, wn_ref, b_ref,
           wfc_ref, bfc_ref, logits_ref, h2_ref):
    agg = (q0_ref[...] + q1_ref[...]) * dinv_ref[...]
    h2 = (jnp.dot(h_ref[...], ws_ref[...], preferred_element_type=jnp.float32)
          + jnp.dot(agg, wn_ref[...], preferred_element_type=jnp.float32)
          + b_ref[...])
    h2_ref[...] = h2
    logits_ref[...] = (
        jnp.dot(h2, wfc_ref[...], preferred_element_type=jnp.float32)
        + bfc_ref[...])

  return pl.pallas_call(
      body,
      grid=(n // bn,),
      in_specs=[
          pl.BlockSpec((bn, d), lambda i: (i, 0)),
          pl.BlockSpec((bn, d), lambda i: (i, 0)),
          pl.BlockSpec((bn, d), lambda i: (i, 0)),
          pl.BlockSpec((bn, d), lambda i: (i, 0)),
          pl.BlockSpec((d, d), lambda i: (0, 0)),
          pl.BlockSpec((d, d), lambda i: (0, 0)),
          pl.BlockSpec((1, d), lambda i: (0, 0)),
          pl.BlockSpec((d, co), lambda i: (0, 0)),
          pl.BlockSpec((1, co), lambda i: (0, 0)),
      ],
      out_specs=[pl.BlockSpec((bn, co), lambda i: (i, 0)),
                 pl.BlockSpec((bn, d), lambda i: (i, 0))],
      out_shape=[jax.ShapeDtypeStruct((n, co), jnp.float32),
                 jax.ShapeDtypeStruct((n, d), jnp.float32)],
  )(h, q0, q1, dinv, w_self, w_neigh, b.reshape(1, d), w_fc,
    b_fc.reshape(1, co))


def kernel(x, edge_index, W_self1, W_neigh1, b1, W_self2, W_neigh2, b2,
           W_fc, b_fc):
  n, d = x.shape
  e = edge_index.shape[1]
  w1 = d + 16
  n2 = ((n + 127) // 128) * 128

  src = edge_index[0].astype(jnp.int32)
  dst = edge_index[1].astype(jnp.int32)
  xpad = jnp.concatenate(
      [x, jnp.ones((n, 1), jnp.float32), jnp.zeros((n, 15), jnp.float32)],
      axis=1)
  zeros1 = jnp.zeros((n2, w1), jnp.float32)
  zeros2 = jnp.zeros((n2, d), jnp.float32)

  p = _make_sc_agg(n2, e, w1)(xpad, src, dst, zeros1)
  h, dinv = _tc_layer1(x, p[:n], p[n2:n2 + n], W_self1, W_neigh1, b1)
  q = _make_sc_agg(n2, e, d)(h, src, dst, zeros2)
  logits, h2 = _tc_layer2(h, q[:n], q[n2:n2 + n], dinv, W_self2, W_neigh2,
                          b2, W_fc, b_fc)
  return (logits, h2)
